# Initial kernel scaffold; baseline (speedup 1.0000x reference)
import jax
import jax.numpy as jnp
from jax.experimental import pallas as pl


def kernel(missing_pattern, tag_table):
    raise NotImplementedError("write your pallas kernel here")


# speedup vs baseline: 1.9339x; 1.9339x over previous
import functools

import jax
import jax.numpy as jnp
from jax import lax
from jax.experimental import pallas as pl
from jax.experimental.pallas import tpu as pltpu
from jax.experimental.pallas import tpu_sc as plsc

_NUM_VIEWS = 6
_VOCAB = 64
_BATCH = 16384
_NC = 2
_NS = 16
_NW = _NC * _NS
_LANES = 16
_BPW = _BATCH // _NW
_OPW = _BPW * _NUM_VIEWS
_GROUPS = _BPW // _LANES


def _make_sc_gather():
    mesh = plsc.VectorSubcoreMesh(core_axis_name="c", subcore_axis_name="s")

    @functools.partial(
        pl.kernel,
        mesh=mesh,
        compiler_params=pltpu.CompilerParams(needs_layout_passes=False),
        out_type=jax.ShapeDtypeStruct((_BATCH * _NUM_VIEWS,), jnp.float32),
        scratch_types=[
            pltpu.VMEM((_BPW,), jnp.int32),
            pltpu.VMEM((_VOCAB * _NUM_VIEWS,), jnp.float32),
            pltpu.VMEM((_OPW,), jnp.float32),
        ],
    )
    def gather_kernel(idx_hbm, tab_hbm, out_hbm, idx_v, tab_v, out_v):
        wid = lax.axis_index("s") * _NC + lax.axis_index("c")
        base = wid * _BPW
        pltpu.sync_copy(tab_hbm, tab_v)
        pltpu.sync_copy(idx_hbm.at[pl.ds(base, _BPW)], idx_v)
        lane6 = lax.iota(jnp.int32, _LANES) * _NUM_VIEWS

        def body(g, carry):
            ids = idx_v[pl.ds(g * _LANES, _LANES)]
            src = ids * _NUM_VIEWS
            dst = lane6 + g * (_LANES * _NUM_VIEWS)
            for d in range(_NUM_VIEWS):
                vals = plsc.load_gather(tab_v, [src + d])
                plsc.store_scatter(out_v, [dst + d], vals)
            return carry

        lax.fori_loop(0, _GROUPS, body, 0)
        pltpu.sync_copy(out_v, out_hbm.at[pl.ds(base * _NUM_VIEWS, _OPW)])

    return gather_kernel


_SC_GATHER = _make_sc_gather()


def kernel(missing_pattern, tag_table):
    idx = missing_pattern.astype(jnp.int32)
    tab = tag_table.reshape(-1).astype(jnp.float32)
    flat = _SC_GATHER(idx, tab)
    return flat.reshape(_BATCH, _NUM_VIEWS)
